# baseline (speedup 1.0000x reference)
import functools

import jax
import jax.numpy as jnp
from jax.experimental import pallas as pl
from jax.experimental.pallas import tpu as pltpu

LATENT = 32
HIDDEN = 64
DEC_HIDDEN = 64
SOFTPLUS_BETA = 100.0
PACK = 4
XSTRIDE = 8


def _softplus_beta_bf(v):
    bv = SOFTPLUS_BETA * v
    safe = jnp.log1p(jnp.exp(jnp.minimum(bv, 20.0))) / SOFTPLUS_BETA
    return jnp.maximum(v, safe)


def _prep_kernel(wpos_p_ref, wpos_n_ref, bpos_ref, w0_ref, b0_ref,
                 w1a_ref, w1p_ref, b1_ref, w2a_ref, w2p_ref, b2_ref,
                 w3a_ref, w3p_ref, b3_ref, wm_ref, bm_ref, ws_ref, bs_ref,
                 dw1x_ref, dw1z_ref, db1_ref, dw2_ref, db2_ref, dw3_ref,
                 o_wpos, o_w0, o_b0, o_w1a, o_w1p, o_b1, o_w2a, o_w2p, o_b2,
                 o_w3a, o_w3p, o_b3, o_wms, o_bms,
                 o_dx, o_dz, o_db1, o_w2d, o_db2, o_w3r, o_sm):
    bf16 = jnp.bfloat16
    H, L, Hd = HIDDEN, LATENT, DEC_HIDDEN
    wpc = jnp.concatenate([wpos_p_ref[...], wpos_n_ref[...]], axis=0)

    o_wpos[...] = jnp.zeros_like(o_wpos)
    o_w0[...] = jnp.zeros_like(o_w0)
    o_wms[...] = jnp.zeros_like(o_wms)
    o_dx[...] = jnp.zeros_like(o_dx)
    o_dz[...] = jnp.zeros_like(o_dz)
    o_w2d[...] = jnp.zeros_like(o_w2d)
    o_w3r[...] = jnp.zeros_like(o_w3r)
    o_sm[...] = jnp.zeros_like(o_sm)
    for oa, op in ((o_w1a, o_w1p), (o_w2a, o_w2p), (o_w3a, o_w3p)):
        oa[...] = jnp.zeros_like(oa)
        op[...] = jnp.zeros_like(op)

    dw3_row = jnp.transpose(dw3_ref[...])
    for e in range(PACK):
        o_wpos[XSTRIDE * e:XSTRIDE * e + 6, 2 * H * e:2 * H * (e + 1)] = (
            wpc.astype(bf16))
        o_wpos[XSTRIDE * e + 6:XSTRIDE * e + 7,
               2 * H * e:2 * H * (e + 1)] = bpos_ref[...].astype(bf16)
        o_w0[2 * H * e:2 * H * (e + 1), H * e:H * (e + 1)] = (
            w0_ref[...].astype(bf16))
        for oa, op, wa_r, wp_r in ((o_w1a, o_w1p, w1a_ref, w1p_ref),
                                   (o_w2a, o_w2p, w2a_ref, w2p_ref),
                                   (o_w3a, o_w3p, w3a_ref, w3p_ref)):
            oa[H * e:H * (e + 1), H * e:H * (e + 1)] = wa_r[...].astype(bf16)
            op[H * e:H * (e + 1), H * e:H * (e + 1)] = wp_r[...].astype(bf16)
        o_wms[H * e:H * (e + 1), L * e:L * (e + 1)] = wm_ref[...].astype(bf16)
        o_wms[H * e:H * (e + 1),
              PACK * L + L * e:PACK * L + L * (e + 1)] = (
            ws_ref[...].astype(bf16))
        o_bms[:, L * e:L * (e + 1)] = bm_ref[...]
        o_bms[:, PACK * L + L * e:PACK * L + L * (e + 1)] = bs_ref[...]
        o_b0[:, H * e:H * (e + 1)] = b0_ref[...]
        o_b1[:, H * e:H * (e + 1)] = b1_ref[...]
        o_b2[:, H * e:H * (e + 1)] = b2_ref[...]
        o_b3[:, H * e:H * (e + 1)] = b3_ref[...]
        o_dx[Hd * e:Hd * (e + 1), 4 * e:4 * e + 3] = dw1x_ref[...].astype(bf16)
        o_dz[Hd * e:Hd * (e + 1), L * e:L * (e + 1)] = dw1z_ref[...]
        o_db1[Hd * e:Hd * (e + 1), :] = db1_ref[...]
        o_w2d[Hd * e:Hd * (e + 1), Hd * e:Hd * (e + 1)] = (
            dw2_ref[...].astype(bf16))
        o_db2[Hd * e:Hd * (e + 1), :] = db2_ref[...]
        o_w3r[e:e + 1, Hd * e:Hd * (e + 1)] = dw3_row.astype(bf16)
        o_sm[L * e:L * (e + 1), e:e + 1] = jnp.full((L, 1), 1.0 / L, bf16)


def _pack_weights(wpos_p, wpos_n, bpos, w0, b0, w1a, w1p, b1, w2a, w2p, b2,
                  w3a, w3p, b3, wm, bm, ws, bs,
                  dw1x, dw1z, db1, dw2, db2, dw3):
    f32, bf16 = jnp.float32, jnp.bfloat16
    H, L, Hd = HIDDEN, LATENT, DEC_HIDDEN
    out_shapes = (
        jax.ShapeDtypeStruct((PACK * XSTRIDE, PACK * 2 * H), bf16),
        jax.ShapeDtypeStruct((PACK * 2 * H, PACK * H), bf16),
        jax.ShapeDtypeStruct((1, PACK * H), f32),
        jax.ShapeDtypeStruct((PACK * H, PACK * H), bf16),
        jax.ShapeDtypeStruct((PACK * H, PACK * H), bf16),
        jax.ShapeDtypeStruct((1, PACK * H), f32),
        jax.ShapeDtypeStruct((PACK * H, PACK * H), bf16),
        jax.ShapeDtypeStruct((PACK * H, PACK * H), bf16),
        jax.ShapeDtypeStruct((1, PACK * H), f32),
        jax.ShapeDtypeStruct((PACK * H, PACK * H), bf16),
        jax.ShapeDtypeStruct((PACK * H, PACK * H), bf16),
        jax.ShapeDtypeStruct((1, PACK * H), f32),
        jax.ShapeDtypeStruct((PACK * H, 2 * PACK * L), bf16),
        jax.ShapeDtypeStruct((1, 2 * PACK * L), f32),
        jax.ShapeDtypeStruct((PACK * Hd, PACK * 4), bf16),
        jax.ShapeDtypeStruct((PACK * Hd, PACK * L), f32),
        jax.ShapeDtypeStruct((PACK * Hd, 1), f32),
        jax.ShapeDtypeStruct((PACK * Hd, PACK * Hd), bf16),
        jax.ShapeDtypeStruct((PACK * Hd, 1), f32),
        jax.ShapeDtypeStruct((PACK, PACK * Hd), bf16),
        jax.ShapeDtypeStruct((PACK * L, PACK), bf16),
    )
    return pl.pallas_call(_prep_kernel, out_shape=out_shapes)(
        wpos_p, wpos_n, bpos, w0, b0, w1a, w1p, b1, w2a, w2p, b2,
        w3a, w3p, b3, wm, bm, ws, bs, dw1x, dw1z, db1, dw2, db2, dw3)


def _fused_kernel(
        x_ref, pt_ref, eps_ref,
        wpos_ref, w0_ref, b0_ref,
        w1a_ref, w1p_ref, b1_ref,
        w2a_ref, w2p_ref, b2_ref,
        w3a_ref, w3p_ref, b3_ref,
        wms_ref, bms_ref,
        dx_ref, dz_ref, db1_ref, w2d_ref, db2_ref, w3r_ref, db3_ref,
        sm_ref,
        sdf_ref, mean_ref, lat_ref, reg_ref,
        lat_sc):
    dot = functools.partial(jnp.dot, preferred_element_type=jnp.float32)
    bf16 = jnp.bfloat16


    lat4p = lat_sc[...]
    latcol = jnp.sum(dz_ref[...] * lat4p, axis=-1, keepdims=True)
    dcol = (latcol + db1_ref[...]).astype(bf16)
    h = dot(dx_ref[...], pt_ref[0])
    hb = _softplus_beta_bf(h.astype(bf16) + dcol)
    h = dot(w2d_ref[...], hb)
    hb = _softplus_beta_bf(h.astype(bf16) + db2_ref[...].astype(bf16))
    sdf_ref[0] = dot(w3r_ref[...], hb) + db3_ref[...]

    x = x_ref[0]
    net = dot(x, wpos_ref[...])
    rb = jnp.maximum(net.astype(bf16), 0.0)
    net = dot(rb, w0_ref[...])
    badd = b0_ref[...]

    def pool_block(net, badd, wa_ref, wp_ref, b_ref):
        rb = jnp.maximum(net.astype(bf16) + badd.astype(bf16), 0.0)
        pooled = jnp.max(rb, axis=0, keepdims=True)
        prow = dot(pooled, wp_ref[...]) + b_ref[...]
        return dot(rb, wa_ref[...]), prow

    net, badd = pool_block(net, badd, w1a_ref, w1p_ref, b1_ref)
    net, badd = pool_block(net, badd, w2a_ref, w2p_ref, b2_ref)
    net, badd = pool_block(net, badd, w3a_ref, w3p_ref, b3_ref)

    pooled = jnp.maximum(jnp.max(net, axis=0, keepdims=True) + badd, 0.0)
    ms = dot(pooled.astype(bf16), wms_ref[...]) + bms_ref[...]
    mean4 = ms[:, :PACK * LATENT]
    std4 = ms[:, PACK * LATENT:]
    lat4 = mean4 + jnp.exp(std4) * eps_ref[0]

    mean_ref[0] = mean4
    lat_ref[0] = lat4
    lat_sc[...] = lat4
    v = jnp.abs(mean4) + jnp.abs(std4 + 1.0)
    reg_ref[0] = dot(v.astype(bf16), sm_ref[...])


def kernel(points_mnfld, normals_mnfld, samples_nonmnfld,
           wpos_p, wpos_n, bpos, w0, b0, w1a, w1p, b1, w2a, w2p, b2,
           w3a, w3p, b3, wm, bm, ws, bs,
           dw1x, dw1z, db1, dw2, db2, dw3, db3, rng):
    f32, bf16 = jnp.float32, jnp.bfloat16
    B, N, _ = points_mnfld.shape
    M = samples_nonmnfld.shape[1]
    G = B // PACK
    L = LATENT

    key = jax.random.wrap_key_data(rng.astype(jnp.uint32))
    eps = jax.random.normal(key, (B, 1, L), f32)
    eps4 = eps.reshape(G, 1, PACK * L)

    pn = jnp.concatenate(
        [points_mnfld.astype(bf16), normals_mnfld.astype(bf16),
         jnp.ones((B, N, 1), bf16), jnp.zeros((B, N, 1), bf16)],
        axis=-1)
    xp = (pn.reshape(G, PACK, N, XSTRIDE).transpose(0, 2, 1, 3)
            .reshape(G, N, PACK * XSTRIDE))
    pt = (samples_nonmnfld[:, :, :4].transpose(0, 2, 1)
          .reshape(G, PACK * 4, M).astype(bf16))
    pts_nm = samples_nonmnfld[:, :, :3]

    weights = _pack_weights(
        wpos_p, wpos_n, bpos, w0, b0, w1a, w1p, b1, w2a, w2p, b2,
        w3a, w3p, b3, wm, bm, ws, bs, dw1x, dw1z, db1, dw2, db2, dw3)
    weights = (*weights[:20], db3, weights[20])

    enc_idx = lambda g: (jnp.minimum(g, G - 1), 0, 0)
    dec_idx = lambda g: (jnp.maximum(g - 1, 0), 0, 0)
    data_specs = [
        pl.BlockSpec((1, N, PACK * XSTRIDE), enc_idx),
        pl.BlockSpec((1, PACK * 4, M), dec_idx),
        pl.BlockSpec((1, 1, PACK * L), enc_idx),
    ]
    weight_specs = [pl.BlockSpec(w.shape, lambda g: (0, 0)) for w in weights]

    out_shapes = (jax.ShapeDtypeStruct((G, PACK, M), f32),
                  jax.ShapeDtypeStruct((G, 1, PACK * L), f32),
                  jax.ShapeDtypeStruct((G, 1, PACK * L), f32),
                  jax.ShapeDtypeStruct((G, 1, PACK), f32))
    out_specs = [pl.BlockSpec((1, PACK, M), dec_idx),
                 pl.BlockSpec((1, 1, PACK * L), enc_idx),
                 pl.BlockSpec((1, 1, PACK * L), enc_idx),
                 pl.BlockSpec((1, 1, PACK), enc_idx)]

    sdf4, mean4, lat4, reg4 = pl.pallas_call(
        _fused_kernel,
        out_shape=out_shapes,
        grid_spec=pltpu.PrefetchScalarGridSpec(
            num_scalar_prefetch=0, grid=(G + 1,),
            in_specs=data_specs + weight_specs,
            out_specs=out_specs,
            scratch_shapes=[pltpu.VMEM((1, PACK * L), f32)]),
        compiler_params=pltpu.CompilerParams(
            dimension_semantics=("arbitrary",)),
    )(xp, pt, eps4, *weights)

    return {
        'points_mnfld': points_mnfld,
        'normals_mnfld': normals_mnfld,
        'samples_nonmnfld': samples_nonmnfld,
        'latent': lat4.reshape(B, L),
        'latent_reg': reg4.reshape(B),
        'q_latent_mean': mean4.reshape(B, L),
        'points_nonmnfld': pts_nm,
        'sdf_nonmnfld': sdf4.reshape(B, M, 1),
    }


# speedup vs baseline: 1.0549x; 1.0549x over previous
import functools

import jax
import jax.numpy as jnp
from jax.experimental import pallas as pl
from jax.experimental.pallas import tpu as pltpu

LATENT = 32
HIDDEN = 64
DEC_HIDDEN = 64
SOFTPLUS_BETA = 100.0
PACK = 4
XSTRIDE = 8


DEC_SCALE = SOFTPLUS_BETA


def _softplus_scaled_bf(u):
    p = jnp.exp(jnp.minimum(u, jnp.bfloat16(20.0)))
    return jnp.maximum(u, jnp.log1p(p))


def _prep_kernel(wpos_p_ref, wpos_n_ref, bpos_ref, w0_ref, b0_ref,
                 w1a_ref, w1p_ref, b1_ref, w2a_ref, w2p_ref, b2_ref,
                 w3a_ref, w3p_ref, b3_ref, wm_ref, bm_ref, ws_ref, bs_ref,
                 dw1x_ref, dw1z_ref, db1_ref, dw2_ref, db2_ref, dw3_ref,
                 o_wpos, o_w0, o_b0, o_w1a, o_w1p, o_b1, o_w2a, o_w2p, o_b2,
                 o_w3a, o_w3p, o_b3, o_wms, o_bms,
                 o_dx, o_dz, o_db1, o_w2d, o_db2, o_w3r, o_sm):
    bf16 = jnp.bfloat16
    H, L, Hd = HIDDEN, LATENT, DEC_HIDDEN
    wpc = jnp.concatenate([wpos_p_ref[...], wpos_n_ref[...]], axis=0)

    o_wpos[...] = jnp.zeros_like(o_wpos)
    o_w0[...] = jnp.zeros_like(o_w0)
    o_wms[...] = jnp.zeros_like(o_wms)
    o_dx[...] = jnp.zeros_like(o_dx)
    o_dz[...] = jnp.zeros_like(o_dz)
    o_w2d[...] = jnp.zeros_like(o_w2d)
    o_w3r[...] = jnp.zeros_like(o_w3r)
    o_sm[...] = jnp.zeros_like(o_sm)
    for oa, op in ((o_w1a, o_w1p), (o_w2a, o_w2p), (o_w3a, o_w3p)):
        oa[...] = jnp.zeros_like(oa)
        op[...] = jnp.zeros_like(op)

    dw3_row = jnp.transpose(dw3_ref[...])
    for e in range(PACK):
        o_wpos[XSTRIDE * e:XSTRIDE * e + 6, 2 * H * e:2 * H * (e + 1)] = (
            wpc.astype(bf16))
        o_wpos[XSTRIDE * e + 6:XSTRIDE * e + 7,
               2 * H * e:2 * H * (e + 1)] = bpos_ref[...].astype(bf16)
        o_w0[2 * H * e:2 * H * (e + 1), H * e:H * (e + 1)] = (
            w0_ref[...].astype(bf16))
        for oa, op, wa_r, wp_r in ((o_w1a, o_w1p, w1a_ref, w1p_ref),
                                   (o_w2a, o_w2p, w2a_ref, w2p_ref),
                                   (o_w3a, o_w3p, w3a_ref, w3p_ref)):
            oa[H * e:H * (e + 1), H * e:H * (e + 1)] = wa_r[...].astype(bf16)
            op[H * e:H * (e + 1), H * e:H * (e + 1)] = wp_r[...].astype(bf16)
        o_wms[H * e:H * (e + 1), L * e:L * (e + 1)] = wm_ref[...].astype(bf16)
        o_wms[H * e:H * (e + 1),
              PACK * L + L * e:PACK * L + L * (e + 1)] = (
            ws_ref[...].astype(bf16))
        o_bms[:, L * e:L * (e + 1)] = bm_ref[...]
        o_bms[:, PACK * L + L * e:PACK * L + L * (e + 1)] = bs_ref[...]
        o_b0[:, H * e:H * (e + 1)] = b0_ref[...]
        o_b1[:, H * e:H * (e + 1)] = b1_ref[...]
        o_b2[:, H * e:H * (e + 1)] = b2_ref[...]
        o_b3[:, H * e:H * (e + 1)] = b3_ref[...]
        o_dx[Hd * e:Hd * (e + 1), 4 * e:4 * e + 3] = (
            (DEC_SCALE * dw1x_ref[...]).astype(bf16))
        o_dz[Hd * e:Hd * (e + 1), L * e:L * (e + 1)] = (
            DEC_SCALE * dw1z_ref[...])
        o_db1[Hd * e:Hd * (e + 1), :] = DEC_SCALE * db1_ref[...]
        o_w2d[Hd * e:Hd * (e + 1), Hd * e:Hd * (e + 1)] = (
            dw2_ref[...].astype(bf16))
        o_db2[Hd * e:Hd * (e + 1), :] = DEC_SCALE * db2_ref[...]
        o_w3r[e:e + 1, Hd * e:Hd * (e + 1)] = (
            (dw3_row / DEC_SCALE).astype(bf16))
        o_sm[L * e:L * (e + 1), e:e + 1] = jnp.full((L, 1), 1.0 / L, bf16)


def _pack_weights(wpos_p, wpos_n, bpos, w0, b0, w1a, w1p, b1, w2a, w2p, b2,
                  w3a, w3p, b3, wm, bm, ws, bs,
                  dw1x, dw1z, db1, dw2, db2, dw3):
    f32, bf16 = jnp.float32, jnp.bfloat16
    H, L, Hd = HIDDEN, LATENT, DEC_HIDDEN
    out_shapes = (
        jax.ShapeDtypeStruct((PACK * XSTRIDE, PACK * 2 * H), bf16),
        jax.ShapeDtypeStruct((PACK * 2 * H, PACK * H), bf16),
        jax.ShapeDtypeStruct((1, PACK * H), f32),
        jax.ShapeDtypeStruct((PACK * H, PACK * H), bf16),
        jax.ShapeDtypeStruct((PACK * H, PACK * H), bf16),
        jax.ShapeDtypeStruct((1, PACK * H), f32),
        jax.ShapeDtypeStruct((PACK * H, PACK * H), bf16),
        jax.ShapeDtypeStruct((PACK * H, PACK * H), bf16),
        jax.ShapeDtypeStruct((1, PACK * H), f32),
        jax.ShapeDtypeStruct((PACK * H, PACK * H), bf16),
        jax.ShapeDtypeStruct((PACK * H, PACK * H), bf16),
        jax.ShapeDtypeStruct((1, PACK * H), f32),
        jax.ShapeDtypeStruct((PACK * H, 2 * PACK * L), bf16),
        jax.ShapeDtypeStruct((1, 2 * PACK * L), f32),
        jax.ShapeDtypeStruct((PACK * Hd, PACK * 4), bf16),
        jax.ShapeDtypeStruct((PACK * Hd, PACK * L), f32),
        jax.ShapeDtypeStruct((PACK * Hd, 1), f32),
        jax.ShapeDtypeStruct((PACK * Hd, PACK * Hd), bf16),
        jax.ShapeDtypeStruct((PACK * Hd, 1), f32),
        jax.ShapeDtypeStruct((PACK, PACK * Hd), bf16),
        jax.ShapeDtypeStruct((PACK * L, PACK), bf16),
    )
    return pl.pallas_call(_prep_kernel, out_shape=out_shapes)(
        wpos_p, wpos_n, bpos, w0, b0, w1a, w1p, b1, w2a, w2p, b2,
        w3a, w3p, b3, wm, bm, ws, bs, dw1x, dw1z, db1, dw2, db2, dw3)


def _fused_kernel(
        x_ref, pt_ref, eps_ref,
        wpos_ref, w0_ref, b0_ref,
        w1a_ref, w1p_ref, b1_ref,
        w2a_ref, w2p_ref, b2_ref,
        w3a_ref, w3p_ref, b3_ref,
        wms_ref, bms_ref,
        dx_ref, dz_ref, db1_ref, w2d_ref, db2_ref, w3r_ref, db3_ref,
        sm_ref,
        sdf_ref, mean_ref, lat_ref, reg_ref):
    dot = functools.partial(jnp.dot, preferred_element_type=jnp.float32)
    bf16 = jnp.bfloat16

    x = x_ref[0]
    net = dot(x, wpos_ref[...])
    rb = jnp.maximum(net.astype(bf16), 0.0)
    net = dot(rb, w0_ref[...])
    badd = b0_ref[...]

    def pool_block(net, badd, wa_ref, wp_ref, b_ref):
        rb = jnp.maximum(net.astype(bf16) + badd.astype(bf16), 0.0)
        pooled = jnp.max(rb, axis=0, keepdims=True)
        prow = dot(pooled, wp_ref[...]) + b_ref[...]
        return dot(rb, wa_ref[...]), prow

    net, badd = pool_block(net, badd, w1a_ref, w1p_ref, b1_ref)
    net, badd = pool_block(net, badd, w2a_ref, w2p_ref, b2_ref)
    net, badd = pool_block(net, badd, w3a_ref, w3p_ref, b3_ref)

    pooled = jnp.maximum(jnp.max(net, axis=0, keepdims=True) + badd, 0.0)
    ms = dot(pooled.astype(bf16), wms_ref[...]) + bms_ref[...]
    mean4 = ms[:, :PACK * LATENT]
    std4 = ms[:, PACK * LATENT:]
    lat4 = mean4 + jnp.exp(std4) * eps_ref[0]

    mean_ref[0] = mean4
    lat_ref[0] = lat4
    v = jnp.abs(mean4) + jnp.abs(std4 + 1.0)
    reg_ref[0] = dot(v.astype(bf16), sm_ref[...])

    latcol = jnp.sum(dz_ref[...] * lat4, axis=-1, keepdims=True)
    dcol = (latcol + db1_ref[...]).astype(bf16)
    u = dot(dx_ref[...], pt_ref[0])
    hb = _softplus_scaled_bf(u.astype(bf16) + dcol)
    u = dot(w2d_ref[...], hb)
    hb = _softplus_scaled_bf(u.astype(bf16) + db2_ref[...].astype(bf16))
    sdf_ref[0] = dot(w3r_ref[...], hb) + db3_ref[...]


def kernel(points_mnfld, normals_mnfld, samples_nonmnfld,
           wpos_p, wpos_n, bpos, w0, b0, w1a, w1p, b1, w2a, w2p, b2,
           w3a, w3p, b3, wm, bm, ws, bs,
           dw1x, dw1z, db1, dw2, db2, dw3, db3, rng):
    f32, bf16 = jnp.float32, jnp.bfloat16
    B, N, _ = points_mnfld.shape
    M = samples_nonmnfld.shape[1]
    G = B // PACK
    L = LATENT

    key = jax.random.wrap_key_data(rng.astype(jnp.uint32))
    eps = jax.random.normal(key, (B, 1, L), f32)
    eps4 = eps.reshape(G, 1, PACK * L)

    pn = jnp.concatenate(
        [points_mnfld.astype(bf16), normals_mnfld.astype(bf16),
         jnp.ones((B, N, 1), bf16), jnp.zeros((B, N, 1), bf16)],
        axis=-1)
    xp = (pn.reshape(G, PACK, N, XSTRIDE).transpose(0, 2, 1, 3)
            .reshape(G, N, PACK * XSTRIDE))
    pt = (samples_nonmnfld[:, :, :4].transpose(0, 2, 1)
          .reshape(G, PACK * 4, M).astype(bf16))
    pts_nm = samples_nonmnfld[:, :, :3]

    weights = _pack_weights(
        wpos_p, wpos_n, bpos, w0, b0, w1a, w1p, b1, w2a, w2p, b2,
        w3a, w3p, b3, wm, bm, ws, bs, dw1x, dw1z, db1, dw2, db2, dw3)
    weights = (*weights[:20], db3, weights[20])

    data_specs = [
        pl.BlockSpec((1, N, PACK * XSTRIDE), lambda g: (g, 0, 0)),
        pl.BlockSpec((1, PACK * 4, M), lambda g: (g, 0, 0)),
        pl.BlockSpec((1, 1, PACK * L), lambda g: (g, 0, 0)),
    ]
    weight_specs = [pl.BlockSpec(w.shape, lambda g: (0, 0)) for w in weights]

    out_shapes = (jax.ShapeDtypeStruct((G, PACK, M), f32),
                  jax.ShapeDtypeStruct((G, 1, PACK * L), f32),
                  jax.ShapeDtypeStruct((G, 1, PACK * L), f32),
                  jax.ShapeDtypeStruct((G, 1, PACK), f32))
    out_specs = [pl.BlockSpec((1, PACK, M), lambda g: (g, 0, 0)),
                 pl.BlockSpec((1, 1, PACK * L), lambda g: (g, 0, 0)),
                 pl.BlockSpec((1, 1, PACK * L), lambda g: (g, 0, 0)),
                 pl.BlockSpec((1, 1, PACK), lambda g: (g, 0, 0))]

    sdf4, mean4, lat4, reg4 = pl.pallas_call(
        _fused_kernel,
        out_shape=out_shapes,
        grid_spec=pltpu.PrefetchScalarGridSpec(
            num_scalar_prefetch=0, grid=(G,),
            in_specs=data_specs + weight_specs,
            out_specs=out_specs),
        compiler_params=pltpu.CompilerParams(
            dimension_semantics=("parallel",)),
    )(xp, pt, eps4, *weights)

    return {
        'points_mnfld': points_mnfld,
        'normals_mnfld': normals_mnfld,
        'samples_nonmnfld': samples_nonmnfld,
        'latent': lat4.reshape(B, L),
        'latent_reg': reg4.reshape(B),
        'q_latent_mean': mean4.reshape(B, L),
        'points_nonmnfld': pts_nm,
        'sdf_nonmnfld': sdf4.reshape(B, M, 1),
    }
